# baseline (speedup 1.0000x reference)
import functools

import jax
import jax.numpy as jnp
from jax import lax
from jax.experimental import pallas as pl
from jax.experimental.pallas import tpu as pltpu
from jax.experimental.pallas import tpu_sc as plsc

_N = 100000
_NPAD = 100096
_E = 6400000
_D = 8
_NC, _NS = 2, 16
_NW = _NC * _NS
_EW = _E // _NW
_W = 1600
_NWIN = _EW // _W
_CH = 64
_NCH = _W // _CH
_RT = _N // _NS


def _sc_body(x_hbm, snd_hbm, rcv_hbm, ed_hbm, z1_hbm, z2_hbm,
             out_hbm, out2_hbm,
             idx_s0, idx_r0, ed_v0, es_v0, rows0,
             idx_s1, idx_r1, ed_v1, es_v1, rows1,
             acc, acc2, gsem0, gsem1, ssem0, ssem1, lsem0, lsem1):
    cid = lax.axis_index("c")
    sid = lax.axis_index("s")
    wid = cid * _NS + sid

    bufs = ((idx_s0, idx_r0, ed_v0, es_v0, rows0, gsem0, ssem0, lsem0),
            (idx_s1, idx_r1, ed_v1, es_v1, rows1, gsem1, ssem1, lsem1))

    pltpu.sync_copy(z1_hbm.at[pl.ds(sid * _RT, _RT)],
                    acc.at[pl.ds(sid * _RT, _RT)])
    @pl.when(sid == 0)
    def _():
        pltpu.sync_copy(z2_hbm, acc2)
    plsc.subcore_barrier()

    def _fire_loads(w, p):
        idx_s, idx_r, ed_v, es_v, rows, gsem, ssem, lsem = bufs[p]
        base = wid * _EW + w * _W
        row0 = base // _CH
        pltpu.async_copy(snd_hbm.at[pl.ds(row0, _NCH)], idx_s, lsem)
        pltpu.async_copy(rcv_hbm.at[pl.ds(row0, _NCH)], idx_r, lsem)
        pltpu.async_copy(ed_hbm.at[pl.ds(base, _W)], ed_v, lsem)

    def _fire_gathers(w, p):
        idx_s, idx_r, ed_v, es_v, rows, gsem, ssem, lsem = bufs[p]
        base = wid * _EW + w * _W
        row0 = base // _CH
        pltpu.make_async_copy(snd_hbm.at[pl.ds(row0, _NCH)], idx_s, lsem).wait()
        pltpu.make_async_copy(rcv_hbm.at[pl.ds(row0, _NCH)], idx_r, lsem).wait()
        pltpu.make_async_copy(ed_hbm.at[pl.ds(base, _W)], ed_v, lsem).wait()
        for c in range(_NCH):
            pltpu.async_copy(x_hbm.at[idx_s.at[c]],
                             rows.at[pl.ds(c * _CH, _CH)], gsem)
        iota16 = lax.iota(jnp.int32, 16)
        col0 = jnp.zeros((16,), jnp.int32)
        for j in range(_W // 16):
            es_v[pl.ds(j * 16, 16)] = plsc.load_gather(
                ed_v, [j * 16 + iota16, col0])

    def _drain_gathers(p):
        idx_s, idx_r, ed_v, es_v, rows, gsem, ssem, lsem = bufs[p]
        for c in range(_NCH):
            pltpu.make_async_copy(x_hbm.at[idx_s.at[c]],
                                  rows.at[pl.ds(c * _CH, _CH)], gsem).wait()

    def _fire_scatters(w, p):
        idx_s, idx_r, ed_v, es_v, rows, gsem, ssem, lsem = bufs[p]
        for c in range(_NCH):
            pltpu.async_copy(rows.at[pl.ds(c * _CH, _CH)],
                             acc.at[idx_r.at[c]], ssem, add=True)
            pltpu.async_copy(es_v.at[pl.ds(c * _CH, _CH)],
                             acc2.at[idx_r.at[c]], ssem, add=True)

    def _drain_scatters(p):
        idx_s, idx_r, ed_v, es_v, rows, gsem, ssem, lsem = bufs[p]
        for c in range(_NCH):
            pltpu.make_async_copy(rows.at[pl.ds(c * _CH, _CH)],
                                  acc.at[idx_r.at[c]], ssem).wait()
            pltpu.make_async_copy(es_v.at[pl.ds(c * _CH, _CH)],
                                  acc2.at[idx_r.at[c]], ssem).wait()

    def _phase(w_cur, cur, w_nxt, nxt, drain_nxt_scatter):
        _drain_gathers(cur)
        _fire_scatters(w_cur, cur)
        if drain_nxt_scatter:
            _drain_scatters(nxt)
        _fire_loads(w_nxt, nxt)
        _fire_gathers(w_nxt, nxt)

    _fire_loads(0, 0)
    _fire_gathers(0, 0)
    _phase(0, 0, 1, 1, False)

    def _body(k, _):
        _phase(2 * k + 1, 1, 2 * k + 2, 0, True)
        _phase(2 * k + 2, 0, 2 * k + 3, 1, True)
        return _
    lax.fori_loop(0, 61, _body, 0)

    _phase(123, 1, 124, 0, True)
    _drain_gathers(0)
    _fire_scatters(124, 0)
    _drain_scatters(1)
    _drain_scatters(0)

    plsc.subcore_barrier()
    pltpu.sync_copy(acc.at[pl.ds(sid * _RT, _RT)],
                    out_hbm.at[cid, pl.ds(sid * _RT, _RT)])
    @pl.when(sid == 0)
    def _():
        pltpu.sync_copy(acc2, out2_hbm.at[cid])


_sc_call = functools.partial(
    pl.kernel,
    out_type=(jax.ShapeDtypeStruct((_NC, _N, _D), jnp.float32),
              jax.ShapeDtypeStruct((_NC, _N), jnp.float32)),
    mesh=plsc.VectorSubcoreMesh(core_axis_name="c", subcore_axis_name="s"),
    scratch_types=[
        pltpu.VMEM((_NCH, _CH), jnp.int32),
        pltpu.VMEM((_NCH, _CH), jnp.int32),
        pltpu.VMEM((_W, 4), jnp.float32),
        pltpu.VMEM((_W,), jnp.float32),
        pltpu.VMEM((_W, _D), jnp.float32),
        pltpu.VMEM((_NCH, _CH), jnp.int32),
        pltpu.VMEM((_NCH, _CH), jnp.int32),
        pltpu.VMEM((_W, 4), jnp.float32),
        pltpu.VMEM((_W,), jnp.float32),
        pltpu.VMEM((_W, _D), jnp.float32),
        pltpu.VMEM_SHARED((_N, _D), jnp.float32),
        pltpu.VMEM_SHARED((_N,), jnp.float32),
        pltpu.SemaphoreType.DMA,
        pltpu.SemaphoreType.DMA,
        pltpu.SemaphoreType.DMA,
        pltpu.SemaphoreType.DMA,
        pltpu.SemaphoreType.DMA,
        pltpu.SemaphoreType.DMA,
    ],
    compiler_params=pltpu.CompilerParams(use_tc_tiling_on_sc=False,
                                         needs_layout_passes=False),
)(_sc_body)


_NB = 4352


def _combine_body(q_ref, acc_ref, se_ref, wm_ref, g_ref, a_ref, c_ref, o_ref):
    s = acc_ref[0] + acc_ref[1]
    m = lax.dot_general(wm_ref[...], s,
                        dimension_numbers=(((1,), (1,)), ((), ())),
                        preferred_element_type=jnp.float32)
    se = se_ref[0:1] + se_ref[1:2]
    o_ref[...] = a_ref[...] * q_ref[...] + m + g_ref[...] * se + c_ref[...]


def _combine(q, acc, se, wm, g, a, c):
    grid = (pl.cdiv(_N, _NB),)
    return pl.pallas_call(
        _combine_body,
        grid=grid,
        in_specs=[
            pl.BlockSpec((8, _NB), lambda i: (0, i)),
            pl.BlockSpec((_NC, _NB, _D), lambda i: (0, i, 0)),
            pl.BlockSpec((_NC, _NB), lambda i: (0, i)),
            pl.BlockSpec((8, _D), lambda i: (0, 0)),
            pl.BlockSpec((8, 1), lambda i: (0, 0)),
            pl.BlockSpec((8, 1), lambda i: (0, 0)),
            pl.BlockSpec((8, 1), lambda i: (0, 0)),
        ],
        out_specs=pl.BlockSpec((8, _NB), lambda i: (0, i)),
        out_shape=jax.ShapeDtypeStruct((8, _N), jnp.float32),
    )(q, acc, se, wm, g, a, c)


def kernel(q, edges, senders, receivers, dt, w_self, w_msg, w_edge,
           w_aux, w_coarse, w_gate, b):
    xt = q.T
    snd2 = senders.astype(jnp.int32).reshape(-1, _CH)
    rcv2 = receivers.astype(jnp.int32).reshape(-1, _CH)
    z1 = jnp.zeros((_N, _D), jnp.float32)
    z2 = jnp.zeros((_N,), jnp.float32)

    acc, se = _sc_call(xt, snd2, rcv2, edges, z1, z2)

    wm = dt[0] * w_msg[:, None] * jnp.eye(8, dtype=jnp.float32)
    g = (dt[0] * w_msg * w_edge)[:, None]
    a = (dt[0] * w_self)[:, None]
    c = (dt[0] * b)[:, None]

    return _combine(q, acc, se, wm, g, a, c)


# speedup vs baseline: 14.9871x; 14.9871x over previous
import functools

import jax
import jax.numpy as jnp
from jax import lax
from jax.experimental import pallas as pl
from jax.experimental.pallas import tpu as pltpu
from jax.experimental.pallas import tpu_sc as plsc

_N = 100000
_NPAD = 100096
_E = 6400000
_D = 8
_NC, _NS = 2, 16
_NW = _NC * _NS
_EW = _E // _NW
_W = 1600
_NWIN = _EW // _W
_CH = 64
_NCH = _W // _CH
_RT = _N // _NS


def _sc_body(x_hbm, snd_hbm, rcv_hbm, es_hbm, z1_hbm, z2_hbm,
             out_hbm, out2_hbm,
             idx_s0, idx_r0, es_v0, rows0, idx_s1, idx_r1, es_v1, rows1,
             acc, acc2, gsem0, gsem1, ssem0, ssem1, lsem0, lsem1):
    cid = lax.axis_index("c")
    sid = lax.axis_index("s")
    wid = cid * _NS + sid

    bufs = ((idx_s0, idx_r0, es_v0, rows0, gsem0, ssem0, lsem0),
            (idx_s1, idx_r1, es_v1, rows1, gsem1, ssem1, lsem1))

    pltpu.sync_copy(z1_hbm.at[pl.ds(sid * _RT, _RT)],
                    acc.at[pl.ds(sid * _RT, _RT)])
    @pl.when(sid == 0)
    def _():
        pltpu.sync_copy(z2_hbm, acc2)
    plsc.subcore_barrier()

    def _fire_loads(w, p):
        idx_s, idx_r, es_v, rows, gsem, ssem, lsem = bufs[p]
        base = wid * _EW + w * _W
        row0 = base // _CH
        pltpu.async_copy(snd_hbm.at[pl.ds(row0, _NCH)], idx_s, lsem)
        pltpu.async_copy(rcv_hbm.at[pl.ds(row0, _NCH)], idx_r, lsem)
        pltpu.async_copy(es_hbm.at[pl.ds(base, _W)], es_v, lsem)

    def _fire_gathers(w, p):
        idx_s, idx_r, es_v, rows, gsem, ssem, lsem = bufs[p]
        base = wid * _EW + w * _W
        row0 = base // _CH
        pltpu.make_async_copy(snd_hbm.at[pl.ds(row0, _NCH)], idx_s, lsem).wait()
        pltpu.make_async_copy(rcv_hbm.at[pl.ds(row0, _NCH)], idx_r, lsem).wait()
        pltpu.make_async_copy(es_hbm.at[pl.ds(base, _W)], es_v, lsem).wait()
        for c in range(_NCH):
            pltpu.async_copy(x_hbm.at[idx_s.at[c]],
                             rows.at[pl.ds(c * _CH, _CH)], gsem)

    def _drain_gathers(p):
        idx_s, idx_r, es_v, rows, gsem, ssem, lsem = bufs[p]
        for c in range(_NCH):
            pltpu.make_async_copy(x_hbm.at[idx_s.at[c]],
                                  rows.at[pl.ds(c * _CH, _CH)], gsem).wait()

    def _fire_scatters(w, p):
        idx_s, idx_r, es_v, rows, gsem, ssem, lsem = bufs[p]
        for c in range(_NCH):
            pltpu.async_copy(rows.at[pl.ds(c * _CH, _CH)],
                             acc.at[idx_r.at[c]], ssem, add=True)
            pltpu.async_copy(es_v.at[pl.ds(c * _CH, _CH)],
                             acc2.at[idx_r.at[c]], ssem, add=True)

    def _drain_scatters(p):
        idx_s, idx_r, es_v, rows, gsem, ssem, lsem = bufs[p]
        for c in range(_NCH):
            pltpu.make_async_copy(rows.at[pl.ds(c * _CH, _CH)],
                                  acc.at[idx_r.at[c]], ssem).wait()
            pltpu.make_async_copy(es_v.at[pl.ds(c * _CH, _CH)],
                                  acc2.at[idx_r.at[c]], ssem).wait()

    def _phase(w_cur, cur, w_nxt, nxt, drain_nxt_scatter):
        _drain_gathers(cur)
        _fire_scatters(w_cur, cur)
        if drain_nxt_scatter:
            _drain_scatters(nxt)
        _fire_loads(w_nxt, nxt)
        _fire_gathers(w_nxt, nxt)

    _fire_loads(0, 0)
    _fire_gathers(0, 0)
    _phase(0, 0, 1, 1, False)

    def _body(k, _):
        _phase(2 * k + 1, 1, 2 * k + 2, 0, True)
        _phase(2 * k + 2, 0, 2 * k + 3, 1, True)
        return _
    lax.fori_loop(0, 61, _body, 0)

    _phase(123, 1, 124, 0, True)
    _drain_gathers(0)
    _fire_scatters(124, 0)
    _drain_scatters(1)
    _drain_scatters(0)

    plsc.subcore_barrier()
    pltpu.sync_copy(acc.at[pl.ds(sid * _RT, _RT)],
                    out_hbm.at[cid, pl.ds(sid * _RT, _RT)])
    @pl.when(sid == 0)
    def _():
        pltpu.sync_copy(acc2, out2_hbm.at[cid])


_sc_call = functools.partial(
    pl.kernel,
    out_type=(jax.ShapeDtypeStruct((_NC, _N, _D), jnp.float32),
              jax.ShapeDtypeStruct((_NC, _N), jnp.float32)),
    mesh=plsc.VectorSubcoreMesh(core_axis_name="c", subcore_axis_name="s"),
    scratch_types=[
        pltpu.VMEM((_NCH, _CH), jnp.int32),
        pltpu.VMEM((_NCH, _CH), jnp.int32),
        pltpu.VMEM((_W,), jnp.float32),
        pltpu.VMEM((_W, _D), jnp.float32),
        pltpu.VMEM((_NCH, _CH), jnp.int32),
        pltpu.VMEM((_NCH, _CH), jnp.int32),
        pltpu.VMEM((_W,), jnp.float32),
        pltpu.VMEM((_W, _D), jnp.float32),
        pltpu.VMEM_SHARED((_N, _D), jnp.float32),
        pltpu.VMEM_SHARED((_N,), jnp.float32),
        pltpu.SemaphoreType.DMA,
        pltpu.SemaphoreType.DMA,
        pltpu.SemaphoreType.DMA,
        pltpu.SemaphoreType.DMA,
        pltpu.SemaphoreType.DMA,
        pltpu.SemaphoreType.DMA,
    ],
    compiler_params=pltpu.CompilerParams(use_tc_tiling_on_sc=False,
                                         needs_layout_passes=False),
)(_sc_body)


_NB = 4352


def _combine_body(q_ref, acc_ref, se_ref, wm_ref, g_ref, a_ref, c_ref, o_ref):
    s = acc_ref[0] + acc_ref[1]
    m = lax.dot_general(wm_ref[...], s,
                        dimension_numbers=(((1,), (1,)), ((), ())),
                        preferred_element_type=jnp.float32)
    se = se_ref[0:1] + se_ref[1:2]
    o_ref[...] = a_ref[...] * q_ref[...] + m + g_ref[...] * se + c_ref[...]


def _combine(q, acc, se, wm, g, a, c):
    grid = (pl.cdiv(_N, _NB),)
    return pl.pallas_call(
        _combine_body,
        grid=grid,
        in_specs=[
            pl.BlockSpec((8, _NB), lambda i: (0, i)),
            pl.BlockSpec((_NC, _NB, _D), lambda i: (0, i, 0)),
            pl.BlockSpec((_NC, _NB), lambda i: (0, i)),
            pl.BlockSpec((8, _D), lambda i: (0, 0)),
            pl.BlockSpec((8, 1), lambda i: (0, 0)),
            pl.BlockSpec((8, 1), lambda i: (0, 0)),
            pl.BlockSpec((8, 1), lambda i: (0, 0)),
        ],
        out_specs=pl.BlockSpec((8, _NB), lambda i: (0, i)),
        out_shape=jax.ShapeDtypeStruct((8, _N), jnp.float32),
    )(q, acc, se, wm, g, a, c)


def kernel(q, edges, senders, receivers, dt, w_self, w_msg, w_edge,
           w_aux, w_coarse, w_gate, b):
    xt = q.T
    snd2 = senders.astype(jnp.int32).reshape(-1, _CH)
    rcv2 = receivers.astype(jnp.int32).reshape(-1, _CH)
    z1 = jnp.zeros((_N, _D), jnp.float32)
    z2 = jnp.zeros((_N,), jnp.float32)

    es = edges[:, 0]
    acc, se = _sc_call(xt, snd2, rcv2, es, z1, z2)

    wm = dt[0] * w_msg[:, None] * jnp.eye(8, dtype=jnp.float32)
    g = (dt[0] * w_msg * w_edge)[:, None]
    a = (dt[0] * w_self)[:, None]
    c = (dt[0] * b)[:, None]

    return _combine(q, acc, se, wm, g, a, c)


# speedup vs baseline: 16.7295x; 1.1163x over previous
import functools

import jax
import jax.numpy as jnp
from jax import lax
from jax.experimental import pallas as pl
from jax.experimental.pallas import tpu as pltpu
from jax.experimental.pallas import tpu_sc as plsc

_N = 100000
_NPAD = 100096
_E = 6400000
_D = 8
_NC, _NS = 2, 16
_NW = _NC * _NS
_EW = _E // _NW
_W = 1600
_NWIN = _EW // _W
_CH = 64
_NCH = _W // _CH
_RT = _N // _NS


def _sc_body(x_hbm, snd_hbm, rcv_hbm, es_hbm, z1_hbm, z2_hbm,
             out_hbm, out2_hbm,
             idx_s0, idx_r0, es_v0, rows0, idx_s1, idx_r1, es_v1, rows1,
             acc, acc2, gsem0, gsem1, ssem0, ssem1, lsem0, lsem1):
    cid = lax.axis_index("c")
    sid = lax.axis_index("s")
    wid = cid * _NS + sid

    bufs = ((idx_s0, idx_r0, es_v0, rows0, gsem0, ssem0, lsem0),
            (idx_s1, idx_r1, es_v1, rows1, gsem1, ssem1, lsem1))

    pltpu.sync_copy(z1_hbm.at[pl.ds(sid * _RT, _RT)],
                    acc.at[pl.ds(sid * _RT, _RT)])
    @pl.when(sid == 0)
    def _():
        pltpu.sync_copy(z2_hbm, acc2)
    plsc.subcore_barrier()

    def _fire_loads(w, p):
        idx_s, idx_r, es_v, rows, gsem, ssem, lsem = bufs[p]
        base = wid * _EW + w * _W
        pltpu.async_copy(snd_hbm.at[pl.ds(base, _W)], idx_s, lsem)
        pltpu.async_copy(rcv_hbm.at[pl.ds(base, _W)], idx_r, lsem)
        pltpu.async_copy(es_hbm.at[pl.ds(base, _W)], es_v, lsem)

    def _fire_gathers(w, p):
        idx_s, idx_r, es_v, rows, gsem, ssem, lsem = bufs[p]
        base = wid * _EW + w * _W
        pltpu.make_async_copy(snd_hbm.at[pl.ds(base, _W)], idx_s, lsem).wait()
        pltpu.make_async_copy(rcv_hbm.at[pl.ds(base, _W)], idx_r, lsem).wait()
        pltpu.make_async_copy(es_hbm.at[pl.ds(base, _W)], es_v, lsem).wait()
        pltpu.async_copy(x_hbm.at[idx_s], rows, gsem)

    def _drain_gathers(p):
        idx_s, idx_r, es_v, rows, gsem, ssem, lsem = bufs[p]
        pltpu.make_async_copy(x_hbm.at[idx_s], rows, gsem).wait()

    def _fire_scatters(w, p):
        idx_s, idx_r, es_v, rows, gsem, ssem, lsem = bufs[p]
        pltpu.async_copy(rows, acc.at[idx_r], ssem, add=True)
        pltpu.async_copy(es_v, acc2.at[idx_r], ssem, add=True)

    def _drain_scatters(p):
        idx_s, idx_r, es_v, rows, gsem, ssem, lsem = bufs[p]
        pltpu.make_async_copy(rows, acc.at[idx_r], ssem).wait()
        pltpu.make_async_copy(es_v, acc2.at[idx_r], ssem).wait()

    def _phase(w_cur, cur, w_nxt, nxt, drain_nxt_scatter):
        _drain_gathers(cur)
        _fire_scatters(w_cur, cur)
        if drain_nxt_scatter:
            _drain_scatters(nxt)
        _fire_loads(w_nxt, nxt)
        _fire_gathers(w_nxt, nxt)

    _fire_loads(0, 0)
    _fire_gathers(0, 0)
    _phase(0, 0, 1, 1, False)

    def _body(k, _):
        _phase(2 * k + 1, 1, 2 * k + 2, 0, True)
        _phase(2 * k + 2, 0, 2 * k + 3, 1, True)
        return _
    lax.fori_loop(0, 61, _body, 0)

    _phase(123, 1, 124, 0, True)
    _drain_gathers(0)
    _fire_scatters(124, 0)
    _drain_scatters(1)
    _drain_scatters(0)

    plsc.subcore_barrier()
    pltpu.sync_copy(acc.at[pl.ds(sid * _RT, _RT)],
                    out_hbm.at[cid, pl.ds(sid * _RT, _RT)])
    @pl.when(sid == 0)
    def _():
        pltpu.sync_copy(acc2, out2_hbm.at[cid])


_sc_call = functools.partial(
    pl.kernel,
    out_type=(jax.ShapeDtypeStruct((_NC, _N, _D), jnp.float32),
              jax.ShapeDtypeStruct((_NC, _N), jnp.float32)),
    mesh=plsc.VectorSubcoreMesh(core_axis_name="c", subcore_axis_name="s"),
    scratch_types=[
        pltpu.VMEM((_W,), jnp.int32),
        pltpu.VMEM((_W,), jnp.int32),
        pltpu.VMEM((_W,), jnp.float32),
        pltpu.VMEM((_W, _D), jnp.float32),
        pltpu.VMEM((_W,), jnp.int32),
        pltpu.VMEM((_W,), jnp.int32),
        pltpu.VMEM((_W,), jnp.float32),
        pltpu.VMEM((_W, _D), jnp.float32),
        pltpu.VMEM_SHARED((_N, _D), jnp.float32),
        pltpu.VMEM_SHARED((_N,), jnp.float32),
        pltpu.SemaphoreType.DMA,
        pltpu.SemaphoreType.DMA,
        pltpu.SemaphoreType.DMA,
        pltpu.SemaphoreType.DMA,
        pltpu.SemaphoreType.DMA,
        pltpu.SemaphoreType.DMA,
    ],
    compiler_params=pltpu.CompilerParams(use_tc_tiling_on_sc=False,
                                         needs_layout_passes=False),
)(_sc_body)


_NB = 4352


def _combine_body(q_ref, acc_ref, se_ref, wm_ref, g_ref, a_ref, c_ref, o_ref):
    s = acc_ref[0] + acc_ref[1]
    m = lax.dot_general(wm_ref[...], s,
                        dimension_numbers=(((1,), (1,)), ((), ())),
                        preferred_element_type=jnp.float32)
    se = se_ref[0:1] + se_ref[1:2]
    o_ref[...] = a_ref[...] * q_ref[...] + m + g_ref[...] * se + c_ref[...]


def _combine(q, acc, se, wm, g, a, c):
    grid = (pl.cdiv(_N, _NB),)
    return pl.pallas_call(
        _combine_body,
        grid=grid,
        in_specs=[
            pl.BlockSpec((8, _NB), lambda i: (0, i)),
            pl.BlockSpec((_NC, _NB, _D), lambda i: (0, i, 0)),
            pl.BlockSpec((_NC, _NB), lambda i: (0, i)),
            pl.BlockSpec((8, _D), lambda i: (0, 0)),
            pl.BlockSpec((8, 1), lambda i: (0, 0)),
            pl.BlockSpec((8, 1), lambda i: (0, 0)),
            pl.BlockSpec((8, 1), lambda i: (0, 0)),
        ],
        out_specs=pl.BlockSpec((8, _NB), lambda i: (0, i)),
        out_shape=jax.ShapeDtypeStruct((8, _N), jnp.float32),
    )(q, acc, se, wm, g, a, c)


def kernel(q, edges, senders, receivers, dt, w_self, w_msg, w_edge,
           w_aux, w_coarse, w_gate, b):
    xt = q.T
    snd2 = senders.astype(jnp.int32)
    rcv2 = receivers.astype(jnp.int32)
    z1 = jnp.zeros((_N, _D), jnp.float32)
    z2 = jnp.zeros((_N,), jnp.float32)

    es = edges[:, 0]
    acc, se = _sc_call(xt, snd2, rcv2, es, z1, z2)

    wm = dt[0] * w_msg[:, None] * jnp.eye(8, dtype=jnp.float32)
    g = (dt[0] * w_msg * w_edge)[:, None]
    a = (dt[0] * w_self)[:, None]
    c = (dt[0] * b)[:, None]

    return _combine(q, acc, se, wm, g, a, c)


# speedup vs baseline: 17.5529x; 1.0492x over previous
import functools

import jax
import jax.numpy as jnp
from jax import lax
from jax.experimental import pallas as pl
from jax.experimental.pallas import tpu as pltpu
from jax.experimental.pallas import tpu_sc as plsc

_N = 100000
_NPAD = 100096
_E = 6400000
_D = 8
_NC, _NS = 2, 16
_NW = _NC * _NS
_EW = _E // _NW
_W = 2000
_NWIN = _EW // _W
_RT = _N // _NS


def _sc_body(x_hbm, snd_hbm, rcv_hbm, es_hbm, z1_hbm, z2_hbm,
             out_hbm, out2_hbm,
             idx_s0, idx_r0, es_v0, rows0, idx_s1, idx_r1, es_v1, rows1,
             acc, acc2, gsem0, gsem1, ssem0, ssem1, lsem0, lsem1):
    cid = lax.axis_index("c")
    sid = lax.axis_index("s")
    wid = cid * _NS + sid

    bufs = ((idx_s0, idx_r0, es_v0, rows0, gsem0, ssem0, lsem0),
            (idx_s1, idx_r1, es_v1, rows1, gsem1, ssem1, lsem1))

    pltpu.sync_copy(z1_hbm.at[pl.ds(sid * _RT, _RT)],
                    acc.at[pl.ds(sid * _RT, _RT)])
    @pl.when(sid == 0)
    def _():
        pltpu.sync_copy(z2_hbm, acc2)
    plsc.subcore_barrier()

    def _fire_loads(w, p):
        idx_s, idx_r, es_v, rows, gsem, ssem, lsem = bufs[p]
        base = wid * _EW + w * _W
        pltpu.async_copy(snd_hbm.at[pl.ds(base, _W)], idx_s, lsem)
        pltpu.async_copy(rcv_hbm.at[pl.ds(base, _W)], idx_r, lsem)
        pltpu.async_copy(es_hbm.at[pl.ds(base, _W)], es_v, lsem)

    def _fire_gathers(w, p):
        idx_s, idx_r, es_v, rows, gsem, ssem, lsem = bufs[p]
        base = wid * _EW + w * _W
        pltpu.make_async_copy(snd_hbm.at[pl.ds(base, _W)], idx_s, lsem).wait()
        pltpu.make_async_copy(rcv_hbm.at[pl.ds(base, _W)], idx_r, lsem).wait()
        pltpu.make_async_copy(es_hbm.at[pl.ds(base, _W)], es_v, lsem).wait()
        pltpu.async_copy(x_hbm.at[idx_s], rows, gsem)

    def _drain_gathers(p):
        idx_s, idx_r, es_v, rows, gsem, ssem, lsem = bufs[p]
        pltpu.make_async_copy(x_hbm.at[idx_s], rows, gsem).wait()

    def _fire_scatters(w, p):
        idx_s, idx_r, es_v, rows, gsem, ssem, lsem = bufs[p]
        pltpu.async_copy(rows, acc.at[idx_r], ssem, add=True)
        pltpu.async_copy(es_v, acc2.at[idx_r], ssem, add=True)

    def _drain_scatters(p):
        idx_s, idx_r, es_v, rows, gsem, ssem, lsem = bufs[p]
        pltpu.make_async_copy(rows, acc.at[idx_r], ssem).wait()
        pltpu.make_async_copy(es_v, acc2.at[idx_r], ssem).wait()

    def _phase(w_cur, cur, w_nxt, nxt, drain_nxt_scatter):
        _drain_gathers(cur)
        _fire_scatters(w_cur, cur)
        if drain_nxt_scatter:
            _drain_scatters(nxt)
        _fire_loads(w_nxt, nxt)
        _fire_gathers(w_nxt, nxt)

    _fire_loads(0, 0)
    _fire_gathers(0, 0)
    _phase(0, 0, 1, 1, False)

    def _body(k, _):
        _phase(2 * k + 1, 1, 2 * k + 2, 0, True)
        _phase(2 * k + 2, 0, 2 * k + 3, 1, True)
        return _
    lax.fori_loop(0, (_NWIN - 2) // 2, _body, 0)

    _drain_gathers(1)
    _fire_scatters(_NWIN - 1, 1)
    _drain_scatters(0)
    _drain_scatters(1)

    plsc.subcore_barrier()
    pltpu.sync_copy(acc.at[pl.ds(sid * _RT, _RT)],
                    out_hbm.at[cid, pl.ds(sid * _RT, _RT)])
    @pl.when(sid == 0)
    def _():
        pltpu.sync_copy(acc2, out2_hbm.at[cid])


_sc_call = functools.partial(
    pl.kernel,
    out_type=(jax.ShapeDtypeStruct((_NC, _N, _D), jnp.float32),
              jax.ShapeDtypeStruct((_NC, _N), jnp.float32)),
    mesh=plsc.VectorSubcoreMesh(core_axis_name="c", subcore_axis_name="s"),
    scratch_types=[
        pltpu.VMEM((_W,), jnp.int32),
        pltpu.VMEM((_W,), jnp.int32),
        pltpu.VMEM((_W,), jnp.float32),
        pltpu.VMEM((_W, _D), jnp.float32),
        pltpu.VMEM((_W,), jnp.int32),
        pltpu.VMEM((_W,), jnp.int32),
        pltpu.VMEM((_W,), jnp.float32),
        pltpu.VMEM((_W, _D), jnp.float32),
        pltpu.VMEM_SHARED((_N, _D), jnp.float32),
        pltpu.VMEM_SHARED((_N,), jnp.float32),
        pltpu.SemaphoreType.DMA,
        pltpu.SemaphoreType.DMA,
        pltpu.SemaphoreType.DMA,
        pltpu.SemaphoreType.DMA,
        pltpu.SemaphoreType.DMA,
        pltpu.SemaphoreType.DMA,
    ],
    compiler_params=pltpu.CompilerParams(use_tc_tiling_on_sc=False,
                                         needs_layout_passes=False),
)(_sc_body)


_NB = 4352


def _combine_body(q_ref, acc_ref, se_ref, wm_ref, g_ref, a_ref, c_ref, o_ref):
    s = acc_ref[0] + acc_ref[1]
    m = lax.dot_general(wm_ref[...], s,
                        dimension_numbers=(((1,), (1,)), ((), ())),
                        preferred_element_type=jnp.float32)
    se = se_ref[0:1] + se_ref[1:2]
    o_ref[...] = a_ref[...] * q_ref[...] + m + g_ref[...] * se + c_ref[...]


def _combine(q, acc, se, wm, g, a, c):
    grid = (pl.cdiv(_N, _NB),)
    return pl.pallas_call(
        _combine_body,
        grid=grid,
        in_specs=[
            pl.BlockSpec((8, _NB), lambda i: (0, i)),
            pl.BlockSpec((_NC, _NB, _D), lambda i: (0, i, 0)),
            pl.BlockSpec((_NC, _NB), lambda i: (0, i)),
            pl.BlockSpec((8, _D), lambda i: (0, 0)),
            pl.BlockSpec((8, 1), lambda i: (0, 0)),
            pl.BlockSpec((8, 1), lambda i: (0, 0)),
            pl.BlockSpec((8, 1), lambda i: (0, 0)),
        ],
        out_specs=pl.BlockSpec((8, _NB), lambda i: (0, i)),
        out_shape=jax.ShapeDtypeStruct((8, _N), jnp.float32),
    )(q, acc, se, wm, g, a, c)


def kernel(q, edges, senders, receivers, dt, w_self, w_msg, w_edge,
           w_aux, w_coarse, w_gate, b):
    xt = q.T
    snd2 = senders.astype(jnp.int32)
    rcv2 = receivers.astype(jnp.int32)
    z1 = jnp.zeros((_N, _D), jnp.float32)
    z2 = jnp.zeros((_N,), jnp.float32)

    es = edges[:, 0]
    acc, se = _sc_call(xt, snd2, rcv2, es, z1, z2)

    wm = dt[0] * w_msg[:, None] * jnp.eye(8, dtype=jnp.float32)
    g = (dt[0] * w_msg * w_edge)[:, None]
    a = (dt[0] * w_self)[:, None]
    c = (dt[0] * b)[:, None]

    return _combine(q, acc, se, wm, g, a, c)


# speedup vs baseline: 20.1779x; 1.1495x over previous
import functools

import jax
import jax.numpy as jnp
from jax import lax
from jax.experimental import pallas as pl
from jax.experimental.pallas import tpu as pltpu
from jax.experimental.pallas import tpu_sc as plsc

_N = 100000
_NPAD = 100096
_E = 6400000
_D = 8
_NC, _NS = 2, 16
_NW = _NC * _NS
_EW = _E // _NW
_W = 2000
_NWIN = _EW // _W
_RT = _N // _NS


def _sc_body(x_hbm, snd_hbm, rcv_hbm, es_hbm, z1_hbm, z2_hbm,
             out_hbm, out2_hbm,
             idx_s0, idx_r0, es_v0, idx_s1, idx_r1, es_v1,
             idx_s2, idx_r2, es_v2, rows0, rows1,
             acc, acc2, gsem0, gsem1, ssem0, ssem1, lsem0, lsem1, lsem2):
    cid = lax.axis_index("c")
    sid = lax.axis_index("s")
    wid = cid * _NS + sid

    lbufs = ((idx_s0, idx_r0, es_v0, lsem0),
             (idx_s1, idx_r1, es_v1, lsem1),
             (idx_s2, idx_r2, es_v2, lsem2))
    rbufs = ((rows0, gsem0, ssem0), (rows1, gsem1, ssem1))

    pltpu.sync_copy(z1_hbm.at[pl.ds(sid * _RT, _RT)],
                    acc.at[pl.ds(sid * _RT, _RT)])
    @pl.when(sid == 0)
    def _():
        pltpu.sync_copy(z2_hbm, acc2)
    plsc.subcore_barrier()

    def _fire_loads(w, j3):
        idx_s, idx_r, es_v, lsem = lbufs[j3]
        base = wid * _EW + w * _W
        pltpu.async_copy(snd_hbm.at[pl.ds(base, _W)], idx_s, lsem)
        pltpu.async_copy(rcv_hbm.at[pl.ds(base, _W)], idx_r, lsem)
        pltpu.async_copy(es_hbm.at[pl.ds(base, _W)], es_v, lsem)

    def _drain_loads(w, j3):
        idx_s, idx_r, es_v, lsem = lbufs[j3]
        base = wid * _EW + w * _W
        pltpu.make_async_copy(snd_hbm.at[pl.ds(base, _W)], idx_s, lsem).wait()
        pltpu.make_async_copy(rcv_hbm.at[pl.ds(base, _W)], idx_r, lsem).wait()
        pltpu.make_async_copy(es_hbm.at[pl.ds(base, _W)], es_v, lsem).wait()

    def _fire_gather(j2, j3):
        rows, gsem, ssem = rbufs[j2]
        pltpu.async_copy(x_hbm.at[lbufs[j3][0]], rows, gsem)

    def _drain_gather(j2, j3):
        rows, gsem, ssem = rbufs[j2]
        pltpu.make_async_copy(x_hbm.at[lbufs[j3][0]], rows, gsem).wait()

    def _fire_scatters(j2, j3):
        rows, gsem, ssem = rbufs[j2]
        idx_r, es_v = lbufs[j3][1], lbufs[j3][2]
        pltpu.async_copy(rows, acc.at[idx_r], ssem, add=True)
        pltpu.async_copy(es_v, acc2.at[idx_r], ssem, add=True)

    def _drain_scatters(j2, j3):
        rows, gsem, ssem = rbufs[j2]
        idx_r, es_v = lbufs[j3][1], lbufs[j3][2]
        pltpu.make_async_copy(rows, acc.at[idx_r], ssem).wait()
        pltpu.make_async_copy(es_v, acc2.at[idx_r], ssem).wait()

    def _phase(w, j2, j3, drain_prev_s, do_loads, do_next_g):
        _drain_gather(j2, j3)
        _fire_scatters(j2, j3)
        if drain_prev_s:
            _drain_scatters(j2 ^ 1, (j3 + 2) % 3)
        if do_loads:
            _fire_loads(w + 2, (j3 + 2) % 3)
        if do_next_g:
            _drain_loads(w + 1, (j3 + 1) % 3)
            _fire_gather(j2 ^ 1, (j3 + 1) % 3)

    _fire_loads(0, 0)
    _fire_loads(1, 1)
    _drain_loads(0, 0)
    _fire_gather(0, 0)
    _phase(0, 0, 0, False, True, True)
    _phase(1, 1, 1, True, True, True)

    def _body(k, w):
        for j in range(6):
            _phase(w + j, j % 2, (2 + j) % 3, True, True, True)
        return w + 6
    w_end = lax.fori_loop(0, (_NWIN - 4) // 6, _body, 2)

    rem = (_NWIN - 4) % 6 + 2
    for j in range(rem):
        w = w_end + j
        _phase(w, j % 2, (2 + j) % 3, True,
               j < rem - 2, j < rem - 1)
    _drain_scatters((rem - 1) % 2, (2 + rem - 1) % 3)

    plsc.subcore_barrier()
    pltpu.sync_copy(acc.at[pl.ds(sid * _RT, _RT)],
                    out_hbm.at[cid, pl.ds(sid * _RT, _RT)])
    @pl.when(sid == 0)
    def _():
        pltpu.sync_copy(acc2, out2_hbm.at[cid])


_sc_call = functools.partial(
    pl.kernel,
    out_type=(jax.ShapeDtypeStruct((_NC, _N, _D), jnp.float32),
              jax.ShapeDtypeStruct((_NC, _N), jnp.float32)),
    mesh=plsc.VectorSubcoreMesh(core_axis_name="c", subcore_axis_name="s"),
    scratch_types=[
        pltpu.VMEM((_W,), jnp.int32),
        pltpu.VMEM((_W,), jnp.int32),
        pltpu.VMEM((_W,), jnp.float32),
        pltpu.VMEM((_W,), jnp.int32),
        pltpu.VMEM((_W,), jnp.int32),
        pltpu.VMEM((_W,), jnp.float32),
        pltpu.VMEM((_W,), jnp.int32),
        pltpu.VMEM((_W,), jnp.int32),
        pltpu.VMEM((_W,), jnp.float32),
        pltpu.VMEM((_W, _D), jnp.float32),
        pltpu.VMEM((_W, _D), jnp.float32),
        pltpu.VMEM_SHARED((_N, _D), jnp.float32),
        pltpu.VMEM_SHARED((_N,), jnp.float32),
        pltpu.SemaphoreType.DMA,
        pltpu.SemaphoreType.DMA,
        pltpu.SemaphoreType.DMA,
        pltpu.SemaphoreType.DMA,
        pltpu.SemaphoreType.DMA,
        pltpu.SemaphoreType.DMA,
        pltpu.SemaphoreType.DMA,
    ],
    compiler_params=pltpu.CompilerParams(use_tc_tiling_on_sc=False,
                                         needs_layout_passes=False),
)(_sc_body)


_NB = 4352


def _combine_body(q_ref, acc_ref, se_ref, wm_ref, g_ref, a_ref, c_ref, o_ref):
    s = acc_ref[0] + acc_ref[1]
    m = lax.dot_general(wm_ref[...], s,
                        dimension_numbers=(((1,), (1,)), ((), ())),
                        preferred_element_type=jnp.float32)
    se = se_ref[0:1] + se_ref[1:2]
    o_ref[...] = a_ref[...] * q_ref[...] + m + g_ref[...] * se + c_ref[...]


def _combine(q, acc, se, wm, g, a, c):
    grid = (pl.cdiv(_N, _NB),)
    return pl.pallas_call(
        _combine_body,
        grid=grid,
        in_specs=[
            pl.BlockSpec((8, _NB), lambda i: (0, i)),
            pl.BlockSpec((_NC, _NB, _D), lambda i: (0, i, 0)),
            pl.BlockSpec((_NC, _NB), lambda i: (0, i)),
            pl.BlockSpec((8, _D), lambda i: (0, 0)),
            pl.BlockSpec((8, 1), lambda i: (0, 0)),
            pl.BlockSpec((8, 1), lambda i: (0, 0)),
            pl.BlockSpec((8, 1), lambda i: (0, 0)),
        ],
        out_specs=pl.BlockSpec((8, _NB), lambda i: (0, i)),
        out_shape=jax.ShapeDtypeStruct((8, _N), jnp.float32),
    )(q, acc, se, wm, g, a, c)


def kernel(q, edges, senders, receivers, dt, w_self, w_msg, w_edge,
           w_aux, w_coarse, w_gate, b):
    xt = q.T
    snd2 = senders.astype(jnp.int32)
    rcv2 = receivers.astype(jnp.int32)
    z1 = jnp.zeros((_N, _D), jnp.float32)
    z2 = jnp.zeros((_N,), jnp.float32)

    es = edges[:, 0]
    acc, se = _sc_call(xt, snd2, rcv2, es, z1, z2)

    wm = dt[0] * w_msg[:, None] * jnp.eye(8, dtype=jnp.float32)
    g = (dt[0] * w_msg * w_edge)[:, None]
    a = (dt[0] * w_self)[:, None]
    c = (dt[0] * b)[:, None]

    return _combine(q, acc, se, wm, g, a, c)


# speedup vs baseline: 20.1998x; 1.0011x over previous
import functools

import jax
import jax.numpy as jnp
from jax import lax
from jax.experimental import pallas as pl
from jax.experimental.pallas import tpu as pltpu
from jax.experimental.pallas import tpu_sc as plsc

_N = 100000
_NPAD = 100096
_E = 6400000
_D = 8
_NC, _NS = 2, 16
_NW = _NC * _NS
_EW = _E // _NW
_W = 2000
_NWIN = _EW // _W
_RT = _N // _NS


def _sc_body(x_hbm, snd_hbm, rcv_hbm, es_hbm, z1_hbm, z2_hbm,
             out_hbm, out2_hbm,
             idx_s0, idx_r0, es_v0, idx_s1, idx_r1, es_v1,
             idx_s2, idx_r2, es_v2, rows0, rows1,
             acc, acc2, gsem0, gsem1, ssem0, ssem1, lsem0, lsem1, lsem2):
    cid = lax.axis_index("c")
    sid = lax.axis_index("s")
    wid = cid * _NS + sid

    lbufs = ((idx_s0, idx_r0, es_v0, lsem0),
             (idx_s1, idx_r1, es_v1, lsem1),
             (idx_s2, idx_r2, es_v2, lsem2))
    rbufs = ((rows0, gsem0, ssem0), (rows1, gsem1, ssem1))

    pltpu.sync_copy(z1_hbm.at[pl.ds(sid * _RT, _RT)],
                    acc.at[pl.ds(sid * _RT, _RT)])
    @pl.when(sid == 0)
    def _():
        pltpu.sync_copy(z2_hbm, acc2)
    plsc.subcore_barrier()

    def _fire_loads(w, j3):
        idx_s, idx_r, es_v, lsem = lbufs[j3]
        base = wid * _EW + w * _W
        pltpu.async_copy(snd_hbm.at[pl.ds(base, _W)], idx_s, lsem)
        pltpu.async_copy(rcv_hbm.at[pl.ds(base, _W)], idx_r, lsem)
        pltpu.async_copy(es_hbm.at[pl.ds(base, _W)], es_v, lsem)

    def _drain_loads(w, j3):
        idx_s, idx_r, es_v, lsem = lbufs[j3]
        base = wid * _EW + w * _W
        pltpu.make_async_copy(snd_hbm.at[pl.ds(base, _W)], idx_s, lsem).wait()
        pltpu.make_async_copy(rcv_hbm.at[pl.ds(base, _W)], idx_r, lsem).wait()
        pltpu.make_async_copy(es_hbm.at[pl.ds(base, _W)], es_v, lsem).wait()

    def _fire_gather(j2, j3):
        rows, gsem, ssem = rbufs[j2]
        pltpu.async_copy(x_hbm.at[lbufs[j3][0]], rows, gsem)

    def _drain_gather(j2, j3):
        rows, gsem, ssem = rbufs[j2]
        pltpu.make_async_copy(x_hbm.at[lbufs[j3][0]], rows, gsem).wait()

    def _fire_scatters(j2, j3):
        rows, gsem, ssem = rbufs[j2]
        idx_r, es_v = lbufs[j3][1], lbufs[j3][2]
        pltpu.async_copy(rows, acc.at[idx_r], ssem, add=True)
        pltpu.async_copy(es_v, acc2.at[idx_r], ssem, add=True)

    def _drain_scatters(j2, j3):
        rows, gsem, ssem = rbufs[j2]
        idx_r, es_v = lbufs[j3][1], lbufs[j3][2]
        pltpu.make_async_copy(rows, acc.at[idx_r], ssem).wait()
        pltpu.make_async_copy(es_v, acc2.at[idx_r], ssem).wait()

    def _phase(w, j2, j3, drain_prev_s, do_loads, do_next_g):
        _drain_gather(j2, j3)
        _fire_scatters(j2, j3)
        if drain_prev_s:
            _drain_scatters(j2 ^ 1, (j3 + 2) % 3)
        if do_loads:
            _fire_loads(w + 2, (j3 + 2) % 3)
        if do_next_g:
            _drain_loads(w + 1, (j3 + 1) % 3)
            _fire_gather(j2 ^ 1, (j3 + 1) % 3)

    _fire_loads(0, 0)
    _fire_loads(1, 1)
    _drain_loads(0, 0)
    _fire_gather(0, 0)
    _phase(0, 0, 0, False, True, True)
    _phase(1, 1, 1, True, True, True)

    def _body(k, w):
        for j in range(6):
            _phase(w + j, j % 2, (2 + j) % 3, True, True, True)
        return w + 6
    w_end = lax.fori_loop(0, (_NWIN - 4) // 6, _body, 2)

    rem = (_NWIN - 4) % 6 + 2
    for j in range(rem):
        w = w_end + j
        _phase(w, j % 2, (2 + j) % 3, True,
               j < rem - 2, j < rem - 1)
    _drain_scatters((rem - 1) % 2, (2 + rem - 1) % 3)

    plsc.subcore_barrier()
    pltpu.sync_copy(acc.at[pl.ds(sid * _RT, _RT)],
                    out_hbm.at[cid, pl.ds(sid * _RT, _RT)])
    @pl.when(sid == 0)
    def _():
        pltpu.sync_copy(acc2, out2_hbm.at[cid])


_sc_call = functools.partial(
    pl.kernel,
    out_type=(jax.ShapeDtypeStruct((_NC, _N, _D), jnp.float32),
              jax.ShapeDtypeStruct((_NC, _N), jnp.float32)),
    mesh=plsc.VectorSubcoreMesh(core_axis_name="c", subcore_axis_name="s"),
    scratch_types=[
        pltpu.VMEM((_W,), jnp.int32),
        pltpu.VMEM((_W,), jnp.int32),
        pltpu.VMEM((_W,), jnp.float32),
        pltpu.VMEM((_W,), jnp.int32),
        pltpu.VMEM((_W,), jnp.int32),
        pltpu.VMEM((_W,), jnp.float32),
        pltpu.VMEM((_W,), jnp.int32),
        pltpu.VMEM((_W,), jnp.int32),
        pltpu.VMEM((_W,), jnp.float32),
        pltpu.VMEM((_W, _D), jnp.float32),
        pltpu.VMEM((_W, _D), jnp.float32),
        pltpu.VMEM_SHARED((_N, _D), jnp.float32),
        pltpu.VMEM_SHARED((_N,), jnp.float32),
        pltpu.SemaphoreType.DMA,
        pltpu.SemaphoreType.DMA,
        pltpu.SemaphoreType.DMA,
        pltpu.SemaphoreType.DMA,
        pltpu.SemaphoreType.DMA,
        pltpu.SemaphoreType.DMA,
        pltpu.SemaphoreType.DMA,
    ],
    compiler_params=pltpu.CompilerParams(use_tc_tiling_on_sc=False,
                                         needs_layout_passes=False),
)(_sc_body)


_NB = 12800


def _combine_body(q_ref, acc_ref, se_ref, wm_ref, g_ref, a_ref, c_ref, o_ref):
    s = acc_ref[0] + acc_ref[1]
    m = lax.dot_general(wm_ref[...], s,
                        dimension_numbers=(((1,), (1,)), ((), ())),
                        preferred_element_type=jnp.float32)
    se = se_ref[0:1] + se_ref[1:2]
    o_ref[...] = a_ref[...] * q_ref[...] + m + g_ref[...] * se + c_ref[...]


def _combine(q, acc, se, wm, g, a, c):
    grid = (pl.cdiv(_N, _NB),)
    return pl.pallas_call(
        _combine_body,
        grid=grid,
        in_specs=[
            pl.BlockSpec((8, _NB), lambda i: (0, i)),
            pl.BlockSpec((_NC, _NB, _D), lambda i: (0, i, 0)),
            pl.BlockSpec((_NC, _NB), lambda i: (0, i)),
            pl.BlockSpec((8, _D), lambda i: (0, 0)),
            pl.BlockSpec((8, 1), lambda i: (0, 0)),
            pl.BlockSpec((8, 1), lambda i: (0, 0)),
            pl.BlockSpec((8, 1), lambda i: (0, 0)),
        ],
        out_specs=pl.BlockSpec((8, _NB), lambda i: (0, i)),
        out_shape=jax.ShapeDtypeStruct((8, _N), jnp.float32),
    )(q, acc, se, wm, g, a, c)


def kernel(q, edges, senders, receivers, dt, w_self, w_msg, w_edge,
           w_aux, w_coarse, w_gate, b):
    xt = q.T
    snd2 = senders.astype(jnp.int32)
    rcv2 = receivers.astype(jnp.int32)
    z1 = jnp.zeros((_N, _D), jnp.float32)
    z2 = jnp.zeros((_N,), jnp.float32)

    es = edges[:, 0]
    acc, se = _sc_call(xt, snd2, rcv2, es, z1, z2)

    wm = dt[0] * w_msg[:, None] * jnp.eye(8, dtype=jnp.float32)
    g = (dt[0] * w_msg * w_edge)[:, None]
    a = (dt[0] * w_self)[:, None]
    c = (dt[0] * b)[:, None]

    return _combine(q, acc, se, wm, g, a, c)
